# baseline (speedup 1.0000x reference)
import functools

import jax
import jax.numpy as jnp
from jax import lax
from jax.experimental import pallas as pl
from jax.experimental.pallas import tpu as pltpu
from jax.experimental.pallas import tpu_sc as plsc

D_MODEL = 768
N_TOKENS = 4 * 2048
NUM_WORKERS = 32
ROWS_PER_WORKER = N_TOKENS // NUM_WORKERS
CHUNK = 16
NCHUNK = ROWS_PER_WORKER // CHUNK
NBUF = 10
OUT_MERGE = 2
NPAIR = NBUF // OUT_MERGE
RING_ROWS = NBUF * CHUNK

_MESH = plsc.VectorSubcoreMesh(core_axis_name="c", subcore_axis_name="s")


@functools.partial(
    pl.kernel,
    mesh=_MESH,
    out_type=jax.ShapeDtypeStruct((N_TOKENS, D_MODEL), jnp.float32),
    scratch_types=[
        pltpu.VMEM((ROWS_PER_WORKER,), jnp.int32),
        pltpu.VMEM((RING_ROWS, D_MODEL), jnp.float32),
    ]
    + [pltpu.SemaphoreType.DMA] * (NBUF + NPAIR),
)
def _embed_gather(ids_hbm, table_hbm, out_hbm, idx_v, rows_v, *sems):
    gsems, osems = sems[:NBUF], sems[NBUF:]
    wid = lax.axis_index("s") * 2 + lax.axis_index("c")
    base = wid * ROWS_PER_WORKER
    pltpu.sync_copy(ids_hbm.at[pl.ds(base, ROWS_PER_WORKER)], idx_v)

    gathers = [None] * NCHUNK
    outs = {}
    out_waited = set()

    def start_gather(j):
        b = j % NBUF
        gathers[j] = pltpu.async_copy(
            table_hbm.at[idx_v.at[pl.ds(j * CHUNK, CHUNK)]],
            rows_v.at[pl.ds(b * CHUNK, CHUNK)], gsems[b])

    def ensure_out_done(p):
        if p in outs and p not in out_waited:
            outs[p].wait()
            out_waited.add(p)

    for j in range(min(NBUF - 1, NCHUNK)):
        start_gather(j)

    for j in range(NCHUNK):
        gathers[j].wait()
        if j % OUT_MERGE == OUT_MERGE - 1:
            p = j // OUT_MERGE
            b0 = (j - OUT_MERGE + 1) % NBUF
            outs[p] = pltpu.async_copy(
                rows_v.at[pl.ds(b0 * CHUNK, OUT_MERGE * CHUNK)],
                out_hbm.at[pl.ds(base + (j - OUT_MERGE + 1) * CHUNK,
                                 OUT_MERGE * CHUNK)],
                osems[p % NPAIR])
        nj = j + NBUF - 1
        if nj < NCHUNK:
            prev = nj - NBUF
            if prev >= 0:
                ensure_out_done(prev // OUT_MERGE)
            start_gather(nj)

    for p in range(NCHUNK // OUT_MERGE):
        ensure_out_done(p)


def kernel(input_ids, embedding_table):
    b, l = input_ids.shape
    ids = input_ids.astype(jnp.int32).reshape(N_TOKENS)
    flat = _embed_gather(ids, embedding_table)
    return flat.reshape(b, l, D_MODEL)


# speedup vs baseline: 1.0045x; 1.0045x over previous
import functools

import jax
import jax.numpy as jnp
from jax import lax
from jax.experimental import pallas as pl
from jax.experimental.pallas import tpu as pltpu
from jax.experimental.pallas import tpu_sc as plsc

D_MODEL = 768
N_TOKENS = 4 * 2048
NUM_WORKERS = 32
ROWS_PER_WORKER = N_TOKENS // NUM_WORKERS
CHUNK = 16
NCHUNK = ROWS_PER_WORKER // CHUNK
NBUF = 8
OUT_MERGE = 4
NPAIR = NBUF // OUT_MERGE
RING_ROWS = NBUF * CHUNK

_MESH = plsc.VectorSubcoreMesh(core_axis_name="c", subcore_axis_name="s")


@functools.partial(
    pl.kernel,
    mesh=_MESH,
    out_type=jax.ShapeDtypeStruct((N_TOKENS, D_MODEL), jnp.float32),
    scratch_types=[
        pltpu.VMEM((ROWS_PER_WORKER,), jnp.int32),
        pltpu.VMEM((RING_ROWS, D_MODEL), jnp.float32),
    ]
    + [pltpu.SemaphoreType.DMA] * (NBUF + NPAIR),
)
def _embed_gather(ids_hbm, table_hbm, out_hbm, idx_v, rows_v, *sems):
    gsems, osems = sems[:NBUF], sems[NBUF:]
    wid = lax.axis_index("s") * 2 + lax.axis_index("c")
    base = wid * ROWS_PER_WORKER
    pltpu.sync_copy(ids_hbm.at[pl.ds(base, ROWS_PER_WORKER)], idx_v)

    gathers = [None] * NCHUNK
    outs = {}
    out_waited = set()

    def start_gather(j):
        b = j % NBUF
        gathers[j] = pltpu.async_copy(
            table_hbm.at[idx_v.at[pl.ds(j * CHUNK, CHUNK)]],
            rows_v.at[pl.ds(b * CHUNK, CHUNK)], gsems[b])

    def ensure_out_done(p):
        if p in outs and p not in out_waited:
            outs[p].wait()
            out_waited.add(p)

    for j in range(min(NBUF - 1, NCHUNK)):
        start_gather(j)

    for j in range(NCHUNK):
        gathers[j].wait()
        if j % OUT_MERGE == OUT_MERGE - 1:
            p = j // OUT_MERGE
            b0 = (j - OUT_MERGE + 1) % NBUF
            outs[p] = pltpu.async_copy(
                rows_v.at[pl.ds(b0 * CHUNK, OUT_MERGE * CHUNK)],
                out_hbm.at[pl.ds(base + (j - OUT_MERGE + 1) * CHUNK,
                                 OUT_MERGE * CHUNK)],
                osems[p % NPAIR])
        nj = j + NBUF - 1
        if nj < NCHUNK:
            prev = nj - NBUF
            if prev >= 0:
                ensure_out_done(prev // OUT_MERGE)
            start_gather(nj)

    for p in range(NCHUNK // OUT_MERGE):
        ensure_out_done(p)


def kernel(input_ids, embedding_table):
    b, l = input_ids.shape
    ids = input_ids.astype(jnp.int32).reshape(N_TOKENS)
    flat = _embed_gather(ids, embedding_table)
    return flat.reshape(b, l, D_MODEL)


# speedup vs baseline: 1.0223x; 1.0177x over previous
import functools

import jax
import jax.numpy as jnp
from jax import lax
from jax.experimental import pallas as pl
from jax.experimental.pallas import tpu as pltpu
from jax.experimental.pallas import tpu_sc as plsc

B_SZ = 4
L_SZ = 2048
D_MODEL = 768
N_TOKENS = B_SZ * L_SZ
NUM_WORKERS = 32
ROWS_PER_WORKER = N_TOKENS // NUM_WORKERS
W_PER_ROW = L_SZ // ROWS_PER_WORKER
CHUNK = 16
NCHUNK = ROWS_PER_WORKER // CHUNK
NBUF = 8
OUT_MERGE = 2
NPAIR = NBUF // OUT_MERGE
RING_ROWS = NBUF * CHUNK

_MESH = plsc.VectorSubcoreMesh(core_axis_name="c", subcore_axis_name="s")


@functools.partial(
    pl.kernel,
    mesh=_MESH,
    out_type=jax.ShapeDtypeStruct((N_TOKENS, D_MODEL), jnp.float32),
    scratch_types=[
        pltpu.VMEM((ROWS_PER_WORKER,), jnp.int32),
        pltpu.VMEM((RING_ROWS, D_MODEL), jnp.float32),
    ]
    + [pltpu.SemaphoreType.DMA] * (NBUF + NPAIR),
)
def _embed_gather(ids_hbm, table_hbm, out_hbm, idx_v, rows_v, *sems):
    gsems, osems = sems[:NBUF], sems[NBUF:]
    wid = lax.axis_index("s") * 2 + lax.axis_index("c")
    base = wid * ROWS_PER_WORKER
    pltpu.sync_copy(
        ids_hbm.at[wid // W_PER_ROW,
                   pl.ds((wid % W_PER_ROW) * ROWS_PER_WORKER,
                         ROWS_PER_WORKER)],
        idx_v)

    gathers = [None] * NCHUNK
    outs = {}
    out_waited = set()

    def start_gather(j):
        b = j % NBUF
        gathers[j] = pltpu.async_copy(
            table_hbm.at[idx_v.at[pl.ds(j * CHUNK, CHUNK)]],
            rows_v.at[pl.ds(b * CHUNK, CHUNK)], gsems[b])

    def ensure_out_done(p):
        if p in outs and p not in out_waited:
            outs[p].wait()
            out_waited.add(p)

    for j in range(min(NBUF - 1, NCHUNK)):
        start_gather(j)

    for j in range(NCHUNK):
        gathers[j].wait()
        if j % OUT_MERGE == OUT_MERGE - 1:
            p = j // OUT_MERGE
            b0 = (j - OUT_MERGE + 1) % NBUF
            outs[p] = pltpu.async_copy(
                rows_v.at[pl.ds(b0 * CHUNK, OUT_MERGE * CHUNK)],
                out_hbm.at[pl.ds(base + (j - OUT_MERGE + 1) * CHUNK,
                                 OUT_MERGE * CHUNK)],
                osems[p % NPAIR])
        nj = j + NBUF - 1
        if nj < NCHUNK:
            prev = nj - NBUF
            if prev >= 0:
                ensure_out_done(prev // OUT_MERGE)
            start_gather(nj)

    for p in range(NCHUNK // OUT_MERGE):
        ensure_out_done(p)


def kernel(input_ids, embedding_table):
    b, l = input_ids.shape
    flat = _embed_gather(input_ids.astype(jnp.int32), embedding_table)
    return flat.reshape(b, l, D_MODEL)


# speedup vs baseline: 1.0262x; 1.0038x over previous
import functools

import jax
import jax.numpy as jnp
from jax import lax
from jax.experimental import pallas as pl
from jax.experimental.pallas import tpu as pltpu
from jax.experimental.pallas import tpu_sc as plsc

B_SZ = 4
L_SZ = 2048
D_MODEL = 768
N_TOKENS = B_SZ * L_SZ
NUM_WORKERS = 32
ROWS_PER_WORKER = N_TOKENS // NUM_WORKERS
W_PER_ROW = L_SZ // ROWS_PER_WORKER
CHUNK = 16
NCHUNK = ROWS_PER_WORKER // CHUNK
NBUF = 8
OUT_MERGE = 2
NPAIR = NBUF // OUT_MERGE
RING_ROWS = NBUF * CHUNK

_MESH = plsc.VectorSubcoreMesh(core_axis_name="c", subcore_axis_name="s")


@functools.partial(
    pl.kernel,
    mesh=_MESH,
    out_type=jax.ShapeDtypeStruct((N_TOKENS, D_MODEL), jnp.float32),
    scratch_types=[
        pltpu.VMEM((ROWS_PER_WORKER,), jnp.int32),
        pltpu.VMEM((RING_ROWS, D_MODEL), jnp.float32),
    ]
    + [pltpu.SemaphoreType.DMA] * (NBUF + NPAIR + 1),
)
def _embed_gather(ids_hbm, table_hbm, out_hbm, idx_v, rows_v, *sems):
    gsems, osems, isem = sems[:NBUF], sems[NBUF:NBUF + NPAIR], sems[-1]
    wid = lax.axis_index("s") * 2 + lax.axis_index("c")
    base = wid * ROWS_PER_WORKER
    half = ROWS_PER_WORKER // 2
    row = wid // W_PER_ROW
    col = (wid % W_PER_ROW) * ROWS_PER_WORKER
    pltpu.sync_copy(ids_hbm.at[row, pl.ds(col, half)],
                    idx_v.at[pl.ds(0, half)])
    idx_rest = pltpu.async_copy(ids_hbm.at[row, pl.ds(col + half, half)],
                                idx_v.at[pl.ds(half, half)], isem)

    gathers = [None] * NCHUNK
    outs = {}
    out_waited = set()

    def start_gather(j):
        b = j % NBUF
        gathers[j] = pltpu.async_copy(
            table_hbm.at[idx_v.at[pl.ds(j * CHUNK, CHUNK)]],
            rows_v.at[pl.ds(b * CHUNK, CHUNK)], gsems[b])

    def ensure_out_done(p):
        if p in outs and p not in out_waited:
            outs[p].wait()
            out_waited.add(p)

    assert (NBUF - 1) * CHUNK <= ROWS_PER_WORKER // 2
    for j in range(min(NBUF - 1, NCHUNK)):
        start_gather(j)
    idx_rest.wait()

    for j in range(NCHUNK):
        gathers[j].wait()
        if j % OUT_MERGE == OUT_MERGE - 1:
            p = j // OUT_MERGE
            b0 = (j - OUT_MERGE + 1) % NBUF
            outs[p] = pltpu.async_copy(
                rows_v.at[pl.ds(b0 * CHUNK, OUT_MERGE * CHUNK)],
                out_hbm.at[pl.ds(base + (j - OUT_MERGE + 1) * CHUNK,
                                 OUT_MERGE * CHUNK)],
                osems[p % NPAIR])
        nj = j + NBUF - 1
        if nj < NCHUNK:
            prev = nj - NBUF
            if prev >= 0:
                ensure_out_done(prev // OUT_MERGE)
            start_gather(nj)

    for p in range(NCHUNK // OUT_MERGE):
        ensure_out_done(p)


def kernel(input_ids, embedding_table):
    b, l = input_ids.shape
    flat = _embed_gather(input_ids.astype(jnp.int32), embedding_table)
    return flat.reshape(b, l, D_MODEL)
